# Initial kernel scaffold; baseline (speedup 1.0000x reference)
import jax
import jax.numpy as jnp
from jax.experimental import pallas as pl


def kernel(x, geo_x, geo_y):
    raise NotImplementedError("write your pallas kernel here")


# speedup vs baseline: 1.1197x; 1.1197x over previous
import functools

import jax
import jax.numpy as jnp
from jax import lax
from jax.experimental import pallas as pl
from jax.experimental.pallas import tpu as pltpu
from jax.experimental.pallas import tpu_sc as plsc

_N_LOC = 1000000
_BATCH = 4096
_SEQ = 200
_SEQP = 208
_NW = 32
_RPW = _BATCH // _NW
_NHALF = _RPW * _SEQP + 16
_NIDX = 2 * _NHALF
_ROW_STEPS = _SEQP // 16
_HALF_VROWS = _NHALF // 16


def _sc_body(xr_hbm, tab_hbm, out_hbm, xi_v, vals_v, out_v, sem):
    wid = lax.axis_index("c") * 16 + lax.axis_index("s")

    pltpu.sync_copy(xr_hbm.at[wid], xi_v)
    pltpu.async_copy(tab_hbm.at[xi_v], vals_v, sem).wait()

    lane = lax.iota(jnp.int32, 16)
    seam_mask = lane >= 15
    zero16 = jnp.zeros((16,), jnp.float32)

    def diff_sq(o):
        a = vals_v[pl.ds(o, 16)]
        b = vals_v[pl.ds(o + 1, 16)]
        d = a - b
        return d * d

    def inner(t, carry):
        acc, o = carry
        return (acc + diff_sq(o), o + 16)

    def outer(r, carry):
        acc, o = carry
        acc, o = lax.fori_loop(0, _ROW_STEPS - 1, inner, (acc, o))
        acc = acc + jnp.where(seam_mask, zero16, diff_sq(o))
        return (acc, o + 16)

    def half(h, acc):
        acc, _ = lax.fori_loop(0, _RPW, outer, (acc, h * _NHALF))
        return acc

    acc = lax.fori_loop(0, 2, half, zero16)

    out_v[...] = acc
    pltpu.sync_copy(out_v, out_hbm.at[wid])


@functools.partial(
    pl.kernel,
    out_type=jax.ShapeDtypeStruct((_NW, 16), jnp.float32),
    mesh=plsc.VectorSubcoreMesh(core_axis_name="c", subcore_axis_name="s"),
    scratch_types=[
        pltpu.VMEM((_NIDX,), jnp.int32),
        pltpu.VMEM((_NIDX,), jnp.float32),
        pltpu.VMEM((16,), jnp.float32),
        pltpu.SemaphoreType.DMA,
    ],
)
def _sc_distance_partials(xr_hbm, tab_hbm, out_hbm, xi_v, vals_v, out_v, sem):
    _sc_body(xr_hbm, tab_hbm, out_hbm, xi_v, vals_v, out_v, sem)


def kernel(x, geo_x, geo_y):
    tab = jnp.concatenate(
        [jnp.zeros((1,), jnp.float32), geo_x,
         jnp.zeros((1,), jnp.float32), geo_y])
    xi = x.astype(jnp.int32)
    xpad = jnp.concatenate(
        [xi, jnp.broadcast_to(xi[:, -1:], (_BATCH, _SEQP - _SEQ))], axis=1)
    xw = xpad.reshape(_NW, _RPW * _SEQP)
    xw = jnp.concatenate(
        [xw, jnp.broadcast_to(xw[:, -1:], (_NW, 16))], axis=1)
    xr = jnp.concatenate([xw, xw + jnp.int32(_N_LOC + 1)], axis=1)
    partials = _sc_distance_partials(xr, tab)
    total = jnp.sum(partials)
    return total / (jnp.float32(_BATCH * (_SEQ - 1)) * jnp.float32(1e6))
